# baseline (speedup 1.0000x reference)
import functools
import math

import jax
import jax.numpy as jnp
from jax import lax
from jax.experimental import pallas as pl
from jax.experimental.pallas import tpu as pltpu
from jax.experimental.pallas import tpu_sc as plsc

H = 1024
RATIO = 4
SELK = 512
WIN = 256
SCALE = 1.0 / math.sqrt(H // 16)
TILE = 256
BF = jnp.bfloat16
F32 = jnp.float32


def _cp(ndims):
    return pltpu.CompilerParams(dimension_semantics=("parallel",) * ndims)


def _softmax(s):
    m = jnp.max(s, axis=-1, keepdims=True)
    e = jnp.exp(s - m)
    return e / jnp.sum(e, axis=-1, keepdims=True)


def _w_spec(shape):
    return pl.BlockSpec(shape, lambda *a: (0,) * len(shape))


def _row_spec(n):
    return pl.BlockSpec((1, n, H), lambda b, t: (b, t, 0))


def _qkv_score_body(x_ref, wq, bq, wk, bk, wv, bv, ws, bs,
                    q_out, k_out, v_out, s_out, xh_out):
    x = x_ref[0]
    x16 = x.astype(BF)
    xh_out[0] = x16
    q_out[0] = (jnp.dot(x16, wq[...], preferred_element_type=F32)
                + bq[0]).astype(BF)
    k_out[0] = (jnp.dot(x16, wk[...], preferred_element_type=F32)
                + bk[0]).astype(BF)
    v_out[0] = (jnp.dot(x16, wv[...], preferred_element_type=F32)
                + bv[0]).astype(BF)
    s_out[0] = lax.dot_general(ws[...], x, (((1,), (1,)), ((), ())),
                               preferred_element_type=F32) + bs[...]


def _qkv_score(x, Wq, bq, Wk, bk, Wv, bv, Wst, bs):
    B, S, _ = x.shape
    return pl.pallas_call(
        _qkv_score_body,
        grid=(B, S // TILE),
        compiler_params=_cp(2),
        in_specs=[
            _row_spec(TILE),
            _w_spec((H, H)), _w_spec((1, H)),
            _w_spec((H, H)), _w_spec((1, H)),
            _w_spec((H, H)), _w_spec((1, H)),
            _w_spec((1, H)), _w_spec((1, 1)),
        ],
        out_specs=[_row_spec(TILE), _row_spec(TILE), _row_spec(TILE),
                   pl.BlockSpec((1, 1, TILE), lambda b, t: (b, 0, t)),
                   _row_spec(TILE)],
        out_shape=[jax.ShapeDtypeStruct((B, S, H), BF)] * 3 + [
            jax.ShapeDtypeStruct((B, 1, S), F32),
            jax.ShapeDtypeStruct((B, S, H), BF)],
    )(x, Wq, bq, Wk, bk, Wv, bv, Wst, bs)


def _comp_branch_body(blk_ref, wc, bc, wq, bq, wk, bk, wv, bv, o_ref):
    c = (jnp.dot(blk_ref[0], wc[...],
                 preferred_element_type=F32) + bc[0]).astype(BF)
    qc = jnp.dot(c, wq[...], preferred_element_type=F32) + bq[0]
    kc = jnp.dot(c, wk[...], preferred_element_type=F32) + bk[0]
    vc = jnp.dot(c, wv[...], preferred_element_type=F32) + bv[0]
    s = jnp.dot(qc, kc.T, preferred_element_type=F32) * SCALE
    o_ref[0] = jnp.dot(_softmax(s).astype(BF), vc.astype(BF),
                       preferred_element_type=F32).astype(BF)


def _comp_branch(blocks, Wc, bc, Wq, bq, Wk, bk, Wv, bv):
    B, NB, D = blocks.shape
    w = lambda shape: pl.BlockSpec(shape, lambda b: (0,) * len(shape))
    return pl.pallas_call(
        _comp_branch_body,
        grid=(B,),
        compiler_params=_cp(1),
        in_specs=[pl.BlockSpec((1, NB, D), lambda b: (b, 0, 0)),
                  w((D, H)), w((1, H)),
                  w((H, H)), w((1, H)),
                  w((H, H)), w((1, H)),
                  w((H, H)), w((1, H))],
        out_specs=pl.BlockSpec((1, NB, H), lambda b: (b, 0, 0)),
        out_shape=jax.ShapeDtypeStruct((B, NB, H), BF),
    )(blocks, Wc, bc, Wq, bq, Wk, bk, Wv, bv)


def _excl_prefix(f):
    R, S = f.shape
    x = f
    k = 1
    while k < S:
        x = x + jnp.concatenate([jnp.zeros((R, k), f.dtype), x[:, :-k]], axis=1)
        k *= 2
    return x - f


def _sel_topk_body(s_ref, d_ref):
    B = s_ref.shape[0]
    x = s_ref[:, 0, :] + 0.0
    kf = float(SELK)

    u = lax.bitcast_convert_type(x, jnp.int32)
    key = u ^ ((u >> 31) & jnp.int32(0x7FFFFFFF))
    lo0 = jnp.min(key, axis=1, keepdims=True)
    hi0 = jnp.max(key, axis=1, keepdims=True) + 1

    def body(_, lohi):
        lo, hi = lohi
        mid = (lo & hi) + ((lo ^ hi) >> 1)
        cnt = jnp.sum((key >= mid).astype(F32), axis=1, keepdims=True)
        ge = cnt >= kf
        return (jnp.where(ge, mid, lo), jnp.where(ge, hi, mid))

    lo, _ = lax.fori_loop(0, 32, body, (lo0, hi0))

    gt = key > lo
    eq = key == lo
    need = kf - jnp.sum(gt.astype(F32), axis=1, keepdims=True)
    eq_excl = _excl_prefix(eq.astype(F32))
    sel = gt | (eq & (eq_excl < need))
    pos = _excl_prefix(sel.astype(F32)).astype(jnp.int32)
    offs = lax.broadcasted_iota(jnp.int32, (B, x.shape[1]), 0) * SELK
    d_ref[:, 0, :] = jnp.where(sel, pos + offs, jnp.int32(B * SELK))


def _sel_topk(scores):
    B, _, S = scores.shape
    return pl.pallas_call(
        _sel_topk_body,
        in_specs=[pl.BlockSpec((B, 1, S), lambda: (0, 0, 0))],
        out_specs=pl.BlockSpec((B, 1, S), lambda: (0, 0, 0)),
        out_shape=jax.ShapeDtypeStruct((B, 1, S), jnp.int32),
    )(scores)


def _sc_scatter(xh32, dest, nbatch):
    nrows, width = xh32.shape
    info = plsc.get_sparse_core_info()
    nw = info.num_cores * info.num_subcores
    chunk = nrows // nw
    out_rows = nbatch * SELK + 8
    mesh = plsc.VectorSubcoreMesh(core_axis_name="c", subcore_axis_name="s")

    @functools.partial(
        pl.kernel, mesh=mesh,
        out_type=jax.ShapeDtypeStruct((out_rows, width), jnp.int32),
        scratch_types=[
            pltpu.VMEM((chunk,), jnp.int32),
            pltpu.VMEM((chunk, width), jnp.int32),
            pltpu.SemaphoreType.DMA,
        ],
    )
    def k(xh_hbm, dest_hbm, out_hbm, idx_v, rows_v, sem):
        wid = lax.axis_index("s") * info.num_cores + lax.axis_index("c")
        base = wid * chunk
        pltpu.sync_copy(dest_hbm.at[pl.ds(base, chunk)], idx_v)
        pltpu.sync_copy(xh_hbm.at[pl.ds(base, chunk)], rows_v)
        pltpu.async_copy(rows_v, out_hbm.at[idx_v], sem).wait()

    return k(xh32, dest)


def _sel_attn_body(x_ref, wq, bq, wk, bk, wv, bv, o_ref):
    sx = x_ref[0]
    qs = jnp.dot(sx, wq[...], preferred_element_type=F32) + bq[0]
    ks = jnp.dot(sx, wk[...], preferred_element_type=F32) + bk[0]
    vs = jnp.dot(sx, wv[...], preferred_element_type=F32) + bv[0]
    s = jnp.dot(qs.astype(BF), ks.astype(BF).T,
                preferred_element_type=F32) * SCALE
    o_ref[0] = jnp.dot(_softmax(s).astype(BF), vs.astype(BF),
                       preferred_element_type=F32).astype(BF)


def _sel_attn(selx, Wq, bq, Wk, bk, Wv, bv):
    B = selx.shape[0]
    return pl.pallas_call(
        _sel_attn_body,
        grid=(B,),
        compiler_params=_cp(1),
        in_specs=[pl.BlockSpec((1, SELK, H), lambda b: (b, 0, 0)),
                  _w_spec((H, H)), _w_spec((1, H)),
                  _w_spec((H, H)), _w_spec((1, H)),
                  _w_spec((H, H)), _w_spec((1, H))],
        out_specs=pl.BlockSpec((1, SELK, H), lambda b: (b, 0, 0)),
        out_shape=jax.ShapeDtypeStruct((B, SELK, H), BF),
    )(selx, Wq, bq, Wk, bk, Wv, bv)


def _win_attn(qlo, qhi, klo, khi, vlo, vhi):
    q = jnp.concatenate([qlo[0], qhi[0]], axis=0)
    k = jnp.concatenate([klo[0], khi[0]], axis=0)
    v = jnp.concatenate([vlo[0], vhi[0]], axis=0)
    s = jnp.dot(q, k.T, preferred_element_type=F32) * SCALE
    return jnp.dot(_softmax(s).astype(BF), v, preferred_element_type=F32)


def _finish(out, x):
    r = out * 0.5 + x * 0.5
    mu = jnp.mean(r, axis=-1, keepdims=True)
    var = jnp.mean((r - mu) ** 2, axis=-1, keepdims=True)
    return (r - mu) / jnp.sqrt(var + 1e-6)


def _gates(x, wg, bg):
    g = jax.nn.sigmoid(jnp.dot(x, wg[...], preferred_element_type=F32) + bg[0])
    return g / (jnp.sum(g, axis=-1, keepdims=True) + 1e-6)


def _combine_body(qlo, qhi, klo, khi, vlo, vhi, hs_ref, comp_ref, sel_ref,
                  wg, bg, wo1, wo2, wo3, bo, o_ref, acc_ref):
    j = pl.program_id(1)
    x = hs_ref[0]
    g = _gates(x, wg, bg)
    win = _win_attn(qlo, qhi, klo, khi, vlo, vhi)
    acc_ref[...] = jnp.dot((win * g[:, 2:3]).astype(BF), wo3[...],
                           preferred_element_type=F32) + bo[0]

    @pl.when(j < SELK // WIN)
    def _():
        extra = jnp.dot((comp_ref[0].astype(F32) * g[:, 0:1]).astype(BF),
                        wo1[...], preferred_element_type=F32)
        extra += jnp.dot((sel_ref[0].astype(F32) * g[:, 1:2]).astype(BF),
                         wo2[...], preferred_element_type=F32)
        acc_ref[...] += extra

    o_ref[0] = _finish(acc_ref[...], x)


def _combine(hs, q, k, v, comp_out, sel_out, Wg, bg, Wo1, Wo2, Wo3, bo):
    B, S, _ = hs.shape
    HW = WIN // 2
    NJ = S // WIN
    lo = pl.BlockSpec((1, HW, H), lambda b, j: (b, j, 0))
    hi = pl.BlockSpec((1, HW, H), lambda b, j: (b, j + 1, 0))
    tile = _row_spec(WIN)
    cs_tile = pl.BlockSpec((1, WIN, H), lambda b, j: (b, jnp.minimum(j, SELK // WIN - 1), 0))
    return pl.pallas_call(
        _combine_body,
        grid=(B, NJ),
        compiler_params=_cp(2),
        in_specs=[lo, hi, lo, hi, lo, hi, tile, cs_tile, cs_tile,
                  _w_spec((H, 3)), _w_spec((1, 3)),
                  _w_spec((H, H)), _w_spec((H, H)), _w_spec((H, H)),
                  _w_spec((1, H))],
        out_specs=tile,
        out_shape=jax.ShapeDtypeStruct((B, S, H), F32),
        scratch_shapes=[pltpu.VMEM((WIN, H), F32)],
    )(q, q, k, k, v, v, hs, comp_out, sel_out, Wg, bg, Wo1, Wo2, Wo3, bo)


def kernel(hidden_states, Wq, bq, Wk, bk, Wv, bv, Wo, bo, Wg, bg, Wc, bc, Ws, bs):
    B, S, _ = hidden_states.shape
    bq2, bk2, bv2 = bq[None, :], bk[None, :], bv[None, :]
    bs2, bg2, bo2, bc2 = bs[None, :], bg[None, :], bo[None, :], bc[None, :]
    Wst = Ws.T
    Wq16, Wk16, Wv16 = Wq.astype(BF), Wk.astype(BF), Wv.astype(BF)
    Wc16 = Wc.astype(BF)
    Wo16 = Wo.astype(BF)
    Wo1, Wo2, Wo3 = Wo16[:H], Wo16[H:2 * H], Wo16[2 * H:]

    q, k, v, scores, xh = _qkv_score(hidden_states, Wq16, bq2, Wk16, bk2,
                                     Wv16, bv2, Wst, bs2)

    dest = _sel_topk(scores).reshape(B * S)
    xh32 = lax.bitcast_convert_type(xh.reshape(B * S, H // 2, 2), jnp.int32)
    out32 = _sc_scatter(xh32, dest, B)

    blocks = xh.reshape(B, S // RATIO, RATIO * H)
    comp_out = _comp_branch(blocks, Wc16, bc2, Wq16, bq2, Wk16, bk2, Wv16, bv2)

    selx = lax.bitcast_convert_type(out32[:B * SELK], BF).reshape(B, SELK, H)
    sel_out = _sel_attn(selx, Wq16, bq2, Wk16, bk2, Wv16, bv2)

    return _combine(hidden_states, q, k, v, comp_out, sel_out,
                    Wg, bg2, Wo1, Wo2, Wo3, bo2)


# speedup vs baseline: 2.1710x; 2.1710x over previous
import functools
import math

import jax
import jax.numpy as jnp
from jax import lax
from jax.experimental import pallas as pl
from jax.experimental.pallas import tpu as pltpu

H = 1024
RATIO = 4
SELK = 512
WIN = 256
SCALE = 1.0 / math.sqrt(H // 16)
TILE = 256
BF = jnp.bfloat16
F32 = jnp.float32


def _cp(ndims):
    return pltpu.CompilerParams(dimension_semantics=("parallel",) * ndims)


def _softmax(s):
    m = jnp.max(s, axis=-1, keepdims=True)
    e = jnp.exp(s - m)
    return e / jnp.sum(e, axis=-1, keepdims=True)


def _w_spec(shape):
    return pl.BlockSpec(shape, lambda *a: (0,) * len(shape))


def _row_spec(n):
    return pl.BlockSpec((1, n, H), lambda b, t: (b, t, 0))


def _qkv_score_body(nproj_tiles, x_ref, wq, bq, wk, bk, wv, bv, ws, bs,
                    q_out, k_out, v_out, s_out, xh_out):
    t = pl.program_id(1)
    x = x_ref[0]
    x16 = x.astype(BF)
    xh_out[0] = x16
    s_out[0] = lax.dot_general(ws[...], x, (((1,), (1,)), ((), ())),
                               preferred_element_type=F32) + bs[...]

    @pl.when(t < nproj_tiles)
    def _():
        q_out[0] = (jnp.dot(x16, wq[...], preferred_element_type=F32)
                    + bq[0]).astype(BF)
        k_out[0] = (jnp.dot(x16, wk[...], preferred_element_type=F32)
                    + bk[0]).astype(BF)
        v_out[0] = (jnp.dot(x16, wv[...], preferred_element_type=F32)
                    + bv[0]).astype(BF)


def _qkv_score(x, Wq, bq, Wk, bk, Wv, bv, Wst, bs):
    B, S, _ = x.shape
    win_rows = (S // WIN - 1) * (WIN // 2) + WIN
    nproj_tiles = (win_rows + TILE - 1) // TILE
    return pl.pallas_call(
        functools.partial(_qkv_score_body, nproj_tiles),
        grid=(B, S // TILE),
        compiler_params=_cp(2),
        in_specs=[
            _row_spec(TILE),
            _w_spec((H, H)), _w_spec((1, H)),
            _w_spec((H, H)), _w_spec((1, H)),
            _w_spec((H, H)), _w_spec((1, H)),
            _w_spec((1, H)), _w_spec((1, 1)),
        ],
        out_specs=[_row_spec(TILE), _row_spec(TILE), _row_spec(TILE),
                   pl.BlockSpec((1, 1, TILE), lambda b, t: (b, 0, t)),
                   _row_spec(TILE)],
        out_shape=[jax.ShapeDtypeStruct((B, S, H), BF)] * 3 + [
            jax.ShapeDtypeStruct((B, 1, S), F32),
            jax.ShapeDtypeStruct((B, S, H), BF)],
    )(x, Wq, bq, Wk, bk, Wv, bv, Wst, bs)


def _comp_branch_body(blk_ref, wc, bc, wq, bq, wk, bk, wv, bv, o_ref):
    c = (jnp.dot(blk_ref[0], wc[...],
                 preferred_element_type=F32) + bc[0]).astype(BF)
    qc = jnp.dot(c, wq[...], preferred_element_type=F32) + bq[0]
    kc = jnp.dot(c, wk[...], preferred_element_type=F32) + bk[0]
    vc = jnp.dot(c, wv[...], preferred_element_type=F32) + bv[0]
    s = jnp.dot(qc, kc.T, preferred_element_type=F32) * SCALE
    o_ref[0] = jnp.dot(_softmax(s).astype(BF), vc.astype(BF),
                       preferred_element_type=F32).astype(BF)


def _comp_branch(blocks, Wc, bc, Wq, bq, Wk, bk, Wv, bv):
    B, NB, D = blocks.shape
    w = lambda shape: pl.BlockSpec(shape, lambda b: (0,) * len(shape))
    return pl.pallas_call(
        _comp_branch_body,
        grid=(B,),
        compiler_params=_cp(1),
        in_specs=[pl.BlockSpec((1, NB, D), lambda b: (b, 0, 0)),
                  w((D, H)), w((1, H)),
                  w((H, H)), w((1, H)),
                  w((H, H)), w((1, H)),
                  w((H, H)), w((1, H))],
        out_specs=pl.BlockSpec((1, NB, H), lambda b: (b, 0, 0)),
        out_shape=jax.ShapeDtypeStruct((B, NB, H), BF),
    )(blocks, Wc, bc, Wq, bq, Wk, bk, Wv, bv)


def _excl_prefix(f):
    R, S = f.shape
    x = f
    k = 1
    while k < S:
        x = x + jnp.concatenate([jnp.zeros((R, k), f.dtype), x[:, :-k]], axis=1)
        k *= 2
    return x - f


def _sel_branch_body(s_ref, xh_ref, wq, bq, wk, bk, wv, bv, o_ref):
    B = s_ref.shape[0]
    x = s_ref[:, 0, :] + 0.0
    kf = float(SELK)

    u = lax.bitcast_convert_type(x, jnp.int32)
    key = u ^ ((u >> 31) & jnp.int32(0x7FFFFFFF))
    lo0 = jnp.min(key, axis=1, keepdims=True)
    hi0 = jnp.max(key, axis=1, keepdims=True) + 1

    def body(_, lohi):
        lo, hi = lohi
        mid = (lo & hi) + ((lo ^ hi) >> 1)
        cnt = jnp.sum((key >= mid).astype(F32), axis=1, keepdims=True)
        ge = cnt >= kf
        return (jnp.where(ge, mid, lo), jnp.where(ge, hi, mid))

    lo, _ = lax.fori_loop(0, 32, body, (lo0, hi0))

    gt = key > lo
    eq = key == lo
    need = kf - jnp.sum(gt.astype(F32), axis=1, keepdims=True)
    eq_excl = _excl_prefix(eq.astype(F32))
    sel = gt | (eq & (eq_excl < need))
    pos = _excl_prefix(sel.astype(F32)).astype(jnp.int32)
    kk = lax.broadcasted_iota(jnp.int32, (SELK, x.shape[1]), 0)
    for b in range(B):
        p = jnp.where((kk == pos[b:b + 1]) & sel[b:b + 1], 1.0, 0.0).astype(BF)
        sx = jnp.dot(p, xh_ref[b], preferred_element_type=F32).astype(BF)
        qs = jnp.dot(sx, wq[...], preferred_element_type=F32) + bq[0]
        ks = jnp.dot(sx, wk[...], preferred_element_type=F32) + bk[0]
        vs = jnp.dot(sx, wv[...], preferred_element_type=F32) + bv[0]
        s = jnp.dot(qs.astype(BF), ks.astype(BF).T,
                    preferred_element_type=F32) * SCALE
        o_ref[b] = jnp.dot(_softmax(s).astype(BF), vs.astype(BF),
                           preferred_element_type=F32).astype(BF)


def _sel_branch(scores, xh, Wq, bq, Wk, bk, Wv, bv):
    B, S, _ = xh.shape
    return pl.pallas_call(
        _sel_branch_body,
        in_specs=[pl.BlockSpec((B, 1, S), lambda: (0, 0, 0)),
                  pl.BlockSpec((B, S, H), lambda: (0, 0, 0)),
                  _w_spec((H, H)), _w_spec((1, H)),
                  _w_spec((H, H)), _w_spec((1, H)),
                  _w_spec((H, H)), _w_spec((1, H))],
        out_specs=pl.BlockSpec((B, SELK, H), lambda: (0, 0, 0)),
        out_shape=jax.ShapeDtypeStruct((B, SELK, H), BF),
    )(scores, xh, Wq, bq, Wk, bk, Wv, bv)


def _win_attn(qlo, qhi, klo, khi, vlo, vhi):
    q = jnp.concatenate([qlo[0], qhi[0]], axis=0)
    k = jnp.concatenate([klo[0], khi[0]], axis=0)
    v = jnp.concatenate([vlo[0], vhi[0]], axis=0)
    s = jnp.dot(q, k.T, preferred_element_type=F32) * SCALE
    return jnp.dot(_softmax(s).astype(BF), v, preferred_element_type=F32)


def _finish(out, x):
    r = out * 0.5 + x * 0.5
    mu = jnp.mean(r, axis=-1, keepdims=True)
    var = jnp.mean((r - mu) ** 2, axis=-1, keepdims=True)
    return (r - mu) / jnp.sqrt(var + 1e-6)


def _gates(x, wg, bg):
    g = jax.nn.sigmoid(jnp.dot(x, wg[...], preferred_element_type=F32) + bg[0])
    return g / (jnp.sum(g, axis=-1, keepdims=True) + 1e-6)


def _combine_body(qlo, qhi, klo, khi, vlo, vhi, hs_ref, comp_ref, sel_ref,
                  wg, bg, wo1, wo2, wo3, bo, o_ref, acc_ref):
    j = pl.program_id(1)
    x = hs_ref[0]
    g = _gates(x, wg, bg)
    win = _win_attn(qlo, qhi, klo, khi, vlo, vhi)
    acc_ref[...] = jnp.dot((win * g[:, 2:3]).astype(BF), wo3[...],
                           preferred_element_type=F32) + bo[0]

    @pl.when(j < SELK // WIN)
    def _():
        extra = jnp.dot((comp_ref[0].astype(F32) * g[:, 0:1]).astype(BF),
                        wo1[...], preferred_element_type=F32)
        extra += jnp.dot((sel_ref[0].astype(F32) * g[:, 1:2]).astype(BF),
                         wo2[...], preferred_element_type=F32)
        acc_ref[...] += extra

    o_ref[0] = _finish(acc_ref[...], x)


def _combine(hs, q, k, v, comp_out, sel_out, Wg, bg, Wo1, Wo2, Wo3, bo):
    B, S, _ = hs.shape
    HW = WIN // 2
    NJ = S // WIN
    lo = pl.BlockSpec((1, HW, H), lambda b, j: (b, j, 0))
    hi = pl.BlockSpec((1, HW, H), lambda b, j: (b, j + 1, 0))
    tile = _row_spec(WIN)
    cs_tile = pl.BlockSpec((1, WIN, H), lambda b, j: (b, jnp.minimum(j, SELK // WIN - 1), 0))
    return pl.pallas_call(
        _combine_body,
        grid=(B, NJ),
        compiler_params=_cp(2),
        in_specs=[lo, hi, lo, hi, lo, hi, tile, cs_tile, cs_tile,
                  _w_spec((H, 3)), _w_spec((1, 3)),
                  _w_spec((H, H)), _w_spec((H, H)), _w_spec((H, H)),
                  _w_spec((1, H))],
        out_specs=tile,
        out_shape=jax.ShapeDtypeStruct((B, S, H), F32),
        scratch_shapes=[pltpu.VMEM((WIN, H), F32)],
    )(q, q, k, k, v, v, hs, comp_out, sel_out, Wg, bg, Wo1, Wo2, Wo3, bo)


def kernel(hidden_states, Wq, bq, Wk, bk, Wv, bv, Wo, bo, Wg, bg, Wc, bc, Ws, bs):
    B, S, _ = hidden_states.shape
    bq2, bk2, bv2 = bq[None, :], bk[None, :], bv[None, :]
    bs2, bg2, bo2, bc2 = bs[None, :], bg[None, :], bo[None, :], bc[None, :]
    Wst = Ws.T
    Wq16, Wk16, Wv16 = Wq.astype(BF), Wk.astype(BF), Wv.astype(BF)
    Wc16 = Wc.astype(BF)
    Wo16 = Wo.astype(BF)
    Wo1, Wo2, Wo3 = Wo16[:H], Wo16[H:2 * H], Wo16[2 * H:]

    q, k, v, scores, xh = _qkv_score(hidden_states, Wq16, bq2, Wk16, bk2,
                                     Wv16, bv2, Wst, bs2)

    blocks = xh.reshape(B, S // RATIO, RATIO * H)
    comp_out = _comp_branch(blocks, Wc16, bc2, Wq16, bq2, Wk16, bk2, Wv16, bv2)

    sel_out = _sel_branch(scores, xh, Wq16, bq2, Wk16, bk2, Wv16, bv2)

    return _combine(hidden_states, q, k, v, comp_out, sel_out,
                    Wg, bg2, Wo1, Wo2, Wo3, bo2)
